# baseline (speedup 1.0000x reference)
import functools

import jax
import jax.numpy as jnp
from jax import lax
from jax.experimental import pallas as pl
from jax.experimental.pallas import tpu as pltpu
from jax.experimental.pallas import tpu_sc as plsc

B = 1024
CTX = 20
EMB = 64
VOC = 100000

NC = 2
NS = 16
NW = NC * NS
BPW = B // NW
ROWS = BPW * CTX
IDX_CHUNK = 128
NCHUNK = ROWS // IDX_CHUNK

TN = 2048
KSPLIT = 4
SUB = TN // KSPLIT
NBUF = 4
NFULL = VOC // TN
VOC_ALIGNED = (VOC // 128) * 128
TAIL = VOC_ALIGNED - NFULL * TN
TAIL_LAST = TAIL - 3 * SUB
GRID = NFULL + 1


def _sc_pool_body(table_hbm, idx_hbm, out_hbm, idx_v, rows_v, pooled_v, sem):
    wid = lax.axis_index("s") * NC + lax.axis_index("c")

    pltpu.sync_copy(idx_hbm.at[wid], idx_v)

    copies = [
        pltpu.make_async_copy(
            table_hbm.at[idx_v.at[j]],
            rows_v.at[pl.ds(j * IDX_CHUNK, IDX_CHUNK)],
            sem,
        )
        for j in range(NCHUNK)
    ]
    for c in copies:
        c.start()
    for c in copies:
        c.wait()

    inv = jnp.float32(1.0 / CTX)

    def body(b, carry):
        base = b * CTX
        for j in range(EMB // 16):
            acc = jnp.zeros((16,), jnp.float32)
            for c in range(CTX):
                acc = acc + rows_v[base + c, pl.ds(j * 16, 16)]
            pooled_v[b, pl.ds(j * 16, 16)] = acc * inv
        return carry

    lax.fori_loop(0, BPW, body, 0)

    pltpu.sync_copy(pooled_v, out_hbm.at[pl.ds(wid * BPW, BPW)])


def _sc_pool(emb_table, idx3d):
    kern = pl.kernel(
        _sc_pool_body,
        out_type=jax.ShapeDtypeStruct((B, EMB), jnp.float32),
        mesh=plsc.VectorSubcoreMesh(core_axis_name="c", subcore_axis_name="s"),
        scratch_types=[
            pltpu.VMEM((NCHUNK, IDX_CHUNK), jnp.int32),
            pltpu.VMEM((ROWS, EMB), jnp.float32),
            pltpu.VMEM((BPW, EMB), jnp.float32),
            pltpu.SemaphoreType.DMA,
        ],
        compiler_params=pltpu.CompilerParams(use_tc_tiling_on_sc=False),
    )
    return kern(emb_table, idx3d)


def _strip_copy(scratch, o_hbm, sems, buf, step, k, width):
    return pltpu.make_async_copy(
        scratch.at[buf, :, pl.ds(k * SUB, width)],
        o_hbm.at[:, pl.ds(step * TN + k * SUB, width)],
        sems.at[buf, k],
    )


def _mm_body(p_ref, w_ref, o_hbm, scratch, sems):
    i = pl.program_id(0)
    buf = lax.rem(i, NBUF)

    @pl.when(i >= NBUF)
    def _():
        j = i - NBUF
        for k in range(KSPLIT):
            _strip_copy(scratch, o_hbm, sems, buf, j, k, SUB).wait()

    scratch[buf] = lax.dot_general(
        p_ref[...],
        w_ref[...],
        dimension_numbers=(((1,), (1,)), ((), ())),
        preferred_element_type=jnp.float32,
    )

    @pl.when(i < NFULL)
    def _():
        for k in range(KSPLIT):
            _strip_copy(scratch, o_hbm, sems, buf, i, k, SUB).start()

    @pl.when(i == NFULL)
    def _():
        for k in range(KSPLIT - 1):
            _strip_copy(scratch, o_hbm, sems, buf, i, k, SUB).start()
        _strip_copy(scratch, o_hbm, sems, buf, i, KSPLIT - 1, TAIL_LAST).start()

    @pl.when(i == GRID - 1)
    def _():
        for d in range(NBUF - 1):
            j = i - (NBUF - 1) + d
            b2 = lax.rem(j, NBUF)
            for k in range(KSPLIT):
                _strip_copy(scratch, o_hbm, sems, b2, j, k, SUB).wait()
        for k in range(KSPLIT - 1):
            _strip_copy(scratch, o_hbm, sems, buf, i, k, SUB).wait()
        _strip_copy(scratch, o_hbm, sems, buf, i, KSPLIT - 1, TAIL_LAST).wait()


def _tail_body(p_ref, w_ref, oin_ref, o_ref):
    o_ref[...] = lax.dot_general(
        p_ref[...],
        w_ref[...],
        dimension_numbers=(((1,), (1,)), ((), ())),
        preferred_element_type=jnp.float32,
    )


def _tc_tail(pooled, ffw_weight, out_main):
    blk = VOC_ALIGNED // 128
    return pl.pallas_call(
        _tail_body,
        grid=(1,),
        in_specs=[
            pl.BlockSpec((B, EMB), lambda i: (0, 0)),
            pl.BlockSpec((128, EMB), lambda i: (blk, 0)),
            pl.BlockSpec(memory_space=pl.ANY),
        ],
        out_specs=pl.BlockSpec((B, 128), lambda i: (0, blk)),
        out_shape=jax.ShapeDtypeStruct((B, VOC), jnp.float32),
        input_output_aliases={2: 0},
    )(pooled, ffw_weight, out_main)


def _tc_matmul(pooled, ffw_weight):
    return pl.pallas_call(
        _mm_body,
        grid=(GRID,),
        in_specs=[
            pl.BlockSpec((B, EMB), lambda i: (0, 0)),
            pl.BlockSpec((TN, EMB), lambda i: (i, 0)),
        ],
        out_specs=pl.BlockSpec(memory_space=pl.ANY),
        out_shape=jax.ShapeDtypeStruct((B, VOC), jnp.float32),
        scratch_shapes=[
            pltpu.VMEM((NBUF, B, TN), jnp.float32),
            pltpu.SemaphoreType.DMA((NBUF, KSPLIT)),
        ],
        compiler_params=pltpu.CompilerParams(
            dimension_semantics=("arbitrary",),
        ),
    )(pooled, ffw_weight)


def kernel(inpt, emb_table, ffw_weight):
    idx = inpt.astype(jnp.int32).reshape(NW, NCHUNK, IDX_CHUNK)
    pooled = _sc_pool(emb_table, idx)
    out_main = _tc_matmul(pooled, ffw_weight)
    return _tc_tail(pooled, ffw_weight, out_main)


# speedup vs baseline: 1.0088x; 1.0088x over previous
import functools

import jax
import jax.numpy as jnp
from jax import lax
from jax.experimental import pallas as pl
from jax.experimental.pallas import tpu as pltpu
from jax.experimental.pallas import tpu_sc as plsc

B = 1024
CTX = 20
EMB = 64
VOC = 100000

NC = 2
NS = 16
NW = NC * NS
BPW = B // NW
ROWS = BPW * CTX
IDX_CHUNK = 128
NCHUNK = ROWS // IDX_CHUNK

TM = 256
TN = 157 * 128
NI = B // TM
NJ = pl.cdiv(VOC, TN)


def _sc_pool_body(table_hbm, idx_hbm, out_hbm, idx_v, rows_v, pooled_v, sem):
    wid = lax.axis_index("s") * NC + lax.axis_index("c")

    pltpu.sync_copy(idx_hbm.at[wid], idx_v)

    copies = [
        pltpu.make_async_copy(
            table_hbm.at[idx_v.at[j]],
            rows_v.at[pl.ds(j * IDX_CHUNK, IDX_CHUNK)],
            sem,
        )
        for j in range(NCHUNK)
    ]
    for c in copies:
        c.start()
    for c in copies:
        c.wait()

    inv = jnp.float32(1.0 / CTX)

    def body(b, carry):
        base = b * CTX
        for j in range(EMB // 16):
            acc = jnp.zeros((16,), jnp.float32)
            for c in range(CTX):
                acc = acc + rows_v[base + c, pl.ds(j * 16, 16)]
            pooled_v[b, pl.ds(j * 16, 16)] = acc * inv
        return carry

    lax.fori_loop(0, BPW, body, 0)

    pltpu.sync_copy(pooled_v, out_hbm.at[pl.ds(wid * BPW, BPW)])


def _sc_pool(emb_table, idx3d):
    kern = pl.kernel(
        _sc_pool_body,
        out_type=jax.ShapeDtypeStruct((B, EMB), jnp.float32),
        mesh=plsc.VectorSubcoreMesh(core_axis_name="c", subcore_axis_name="s"),
        scratch_types=[
            pltpu.VMEM((NCHUNK, IDX_CHUNK), jnp.int32),
            pltpu.VMEM((ROWS, EMB), jnp.float32),
            pltpu.VMEM((BPW, EMB), jnp.float32),
            pltpu.SemaphoreType.DMA,
        ],
        compiler_params=pltpu.CompilerParams(use_tc_tiling_on_sc=False),
    )
    return kern(emb_table, idx3d)


def _mm_body(p_ref, w_ref, o_ref):
    o_ref[...] = lax.dot_general(
        p_ref[...],
        w_ref[...],
        dimension_numbers=(((1,), (1,)), ((), ())),
        preferred_element_type=jnp.float32,
    )


def _tc_matmul(pooled, ffw_weight):
    return pl.pallas_call(
        _mm_body,
        grid=(NJ, NI),
        in_specs=[
            pl.BlockSpec((TM, EMB), lambda j, i: (i, 0)),
            pl.BlockSpec((TN, EMB), lambda j, i: (j, 0)),
        ],
        out_specs=pl.BlockSpec((TM, TN), lambda j, i: (i, j)),
        out_shape=jax.ShapeDtypeStruct((B, VOC), jnp.float32),
        compiler_params=pltpu.CompilerParams(
            dimension_semantics=("arbitrary", "arbitrary"),
            vmem_limit_bytes=96 * 1024 * 1024,
        ),
    )(pooled, ffw_weight)


def kernel(inpt, emb_table, ffw_weight):
    idx = inpt.astype(jnp.int32).reshape(NW, NCHUNK, IDX_CHUNK)
    pooled = _sc_pool(emb_table, idx)
    return _tc_matmul(pooled, ffw_weight)


# speedup vs baseline: 2.7616x; 2.7374x over previous
import functools

import jax
import jax.numpy as jnp
from jax import lax
from jax.experimental import pallas as pl
from jax.experimental.pallas import tpu as pltpu
from jax.experimental.pallas import tpu_sc as plsc

B = 1024
CTX = 20
EMB = 64
VOC = 100000

NC = 2
NS = 16
NW = NC * NS
BPW = B // NW
ROWS = BPW * CTX
IDX_CHUNK = 128
NCHUNK = ROWS // IDX_CHUNK

TN = 2048
NJ = pl.cdiv(VOC, TN)


def _sc_pool_body(table_hbm, idx_hbm, out_hbm, idx_v, rows_v, pooled_v, sem):
    wid = lax.axis_index("s") * NC + lax.axis_index("c")

    pltpu.sync_copy(idx_hbm.at[wid], idx_v)

    copies = [
        pltpu.make_async_copy(
            table_hbm.at[idx_v.at[j]],
            rows_v.at[pl.ds(j * IDX_CHUNK, IDX_CHUNK)],
            sem,
        )
        for j in range(NCHUNK)
    ]
    for c in copies:
        c.start()
    for c in copies:
        c.wait()

    inv = jnp.float32(1.0 / CTX)

    def body(b, carry):
        base = b * CTX
        for j in range(EMB // 16):
            acc = jnp.zeros((16,), jnp.float32)
            for c in range(CTX):
                acc = acc + rows_v[base + c, pl.ds(j * 16, 16)]
            pooled_v[b, pl.ds(j * 16, 16)] = acc * inv
        return carry

    lax.fori_loop(0, BPW, body, 0)

    pltpu.sync_copy(pooled_v, out_hbm.at[pl.ds(wid * BPW, BPW)])


def _sc_pool(emb_table, idx3d):
    kern = pl.kernel(
        _sc_pool_body,
        out_type=jax.ShapeDtypeStruct((B, EMB), jnp.float32),
        mesh=plsc.VectorSubcoreMesh(core_axis_name="c", subcore_axis_name="s"),
        scratch_types=[
            pltpu.VMEM((NCHUNK, IDX_CHUNK), jnp.int32),
            pltpu.VMEM((ROWS, EMB), jnp.float32),
            pltpu.VMEM((BPW, EMB), jnp.float32),
            pltpu.SemaphoreType.DMA,
        ],
        compiler_params=pltpu.CompilerParams(use_tc_tiling_on_sc=False),
    )
    return kern(emb_table, idx3d)


def _mm_body(w_ref, p_ref, o_ref):
    o_ref[...] = lax.dot_general(
        w_ref[...],
        p_ref[...],
        dimension_numbers=(((0,), (0,)), ((), ())),
        preferred_element_type=jnp.float32,
    )


def _tc_matmul_t(ffw_t, pooled_t):
    return pl.pallas_call(
        _mm_body,
        grid=(NJ,),
        in_specs=[
            pl.BlockSpec((EMB, TN), lambda j: (0, j)),
            pl.BlockSpec((EMB, B), lambda j: (0, 0)),
        ],
        out_specs=pl.BlockSpec((TN, B), lambda j: (j, 0)),
        out_shape=jax.ShapeDtypeStruct((VOC, B), jnp.float32),
        compiler_params=pltpu.CompilerParams(
            dimension_semantics=("arbitrary",),
        ),
    )(ffw_t, pooled_t)


def kernel(inpt, emb_table, ffw_weight):
    idx = inpt.astype(jnp.int32).reshape(NW, NCHUNK, IDX_CHUNK)
    pooled = _sc_pool(emb_table, idx)
    out_t = _tc_matmul_t(ffw_weight.T, pooled.T)
    return out_t.T


# speedup vs baseline: 2.9973x; 1.0853x over previous
import functools

import jax
import jax.numpy as jnp
from jax import lax
from jax.experimental import pallas as pl
from jax.experimental.pallas import tpu as pltpu
from jax.experimental.pallas import tpu_sc as plsc

B = 1024
CTX = 20
EMB = 64
VOC = 100000

NC = 2
NS = 16
NW = NC * NS
BPW = B // NW
ROWS = BPW * CTX
IDX_CHUNK = 128
NCHUNK = ROWS // IDX_CHUNK

TN = 2048
NJ = pl.cdiv(VOC, TN)


CPW = EMB // NW


def _sc_pool_t_body(emb_u, idx_u, out_hbm, idx_v, row_v, out_v, sem):
    wid = lax.axis_index("s") * NC + lax.axis_index("c")
    pltpu.sync_copy(idx_u, idx_v)
    inv = jnp.float32(1.0 / CTX)

    for comp in range(CPW):
        e = wid * CPW + comp
        pltpu.sync_copy(emb_u.at[e], row_v)

        def body(bb, carry):
            acc = jnp.zeros((16,), jnp.float32)
            for c in range(CTX):
                ivec = idx_v[c, pl.ds(bb * 16, 16)]
                acc = acc + plsc.load_gather(row_v, [ivec])
            out_v[pl.ds(bb * 16, 16)] = acc * inv
            return carry

        lax.fori_loop(0, B // 16, body, 0)
        pltpu.sync_copy(out_v, out_hbm.at[e])


def _sc_pool_t(emb_u, idx_u):
    kern = pl.kernel(
        _sc_pool_t_body,
        out_type=jax.ShapeDtypeStruct((EMB, B), jnp.float32),
        mesh=plsc.VectorSubcoreMesh(core_axis_name="c", subcore_axis_name="s"),
        scratch_types=[
            pltpu.VMEM((CTX, B), jnp.int32),
            pltpu.VMEM((VOC,), jnp.float32),
            pltpu.VMEM((B,), jnp.float32),
            pltpu.SemaphoreType.DMA,
        ],
        compiler_params=pltpu.CompilerParams(
            use_tc_tiling_on_sc=False,
            needs_layout_passes=False,
        ),
    )
    return kern(emb_u, idx_u)


def _mm_body(w_ref, p_ref, o_ref):
    o_ref[...] = lax.dot_general(
        w_ref[...],
        p_ref[...],
        dimension_numbers=(((0,), (0,)), ((), ())),
        preferred_element_type=jnp.float32,
    )


def _tc_matmul_t(ffw_t, pooled_t):
    return pl.pallas_call(
        _mm_body,
        grid=(NJ,),
        in_specs=[
            pl.BlockSpec((EMB, TN), lambda j: (0, j)),
            pl.BlockSpec((EMB, B), lambda j: (0, 0)),
        ],
        out_specs=pl.BlockSpec((TN, B), lambda j: (j, 0)),
        out_shape=jax.ShapeDtypeStruct((VOC, B), jnp.float32),
        compiler_params=pltpu.CompilerParams(
            dimension_semantics=("arbitrary",),
        ),
    )(ffw_t, pooled_t)


def kernel(inpt, emb_table, ffw_weight):
    idx_u = inpt.astype(jnp.int32).T
    pooled_t = _sc_pool_t(emb_table.T, idx_u)
    out_t = _tc_matmul_t(ffw_weight.T, pooled_t)
    return out_t.T


# speedup vs baseline: 3.0097x; 1.0041x over previous
import functools

import jax
import jax.numpy as jnp
from jax import lax
from jax.experimental import pallas as pl
from jax.experimental.pallas import tpu as pltpu
from jax.experimental.pallas import tpu_sc as plsc

B = 1024
CTX = 20
EMB = 64
VOC = 100000

NC = 2
NS = 16
NW = NC * NS
BPW = B // NW
ROWS = BPW * CTX
IDX_CHUNK = 128
NCHUNK = ROWS // IDX_CHUNK

TN = 4096
NJ = pl.cdiv(VOC, TN)


CPW = EMB // NW


def _sc_pool_t_body(emb_u, idx_u, out_hbm, idx_v, row_v, out_v, sem):
    wid = lax.axis_index("s") * NC + lax.axis_index("c")
    pltpu.sync_copy(idx_u, idx_v)
    inv = jnp.float32(1.0 / CTX)

    for comp in range(CPW):
        e = wid * CPW + comp
        pltpu.sync_copy(emb_u.at[e], row_v)

        def body(bb, carry):
            acc = jnp.zeros((16,), jnp.float32)
            for c in range(CTX):
                ivec = idx_v[c, pl.ds(bb * 16, 16)]
                acc = acc + plsc.load_gather(row_v, [ivec])
            out_v[pl.ds(bb * 16, 16)] = acc * inv
            return carry

        lax.fori_loop(0, B // 16, body, 0)
        pltpu.sync_copy(out_v, out_hbm.at[e])


def _sc_pool_t(emb_u, idx_u):
    kern = pl.kernel(
        _sc_pool_t_body,
        out_type=jax.ShapeDtypeStruct((EMB, B), jnp.float32),
        mesh=plsc.VectorSubcoreMesh(core_axis_name="c", subcore_axis_name="s"),
        scratch_types=[
            pltpu.VMEM((CTX, B), jnp.int32),
            pltpu.VMEM((VOC,), jnp.float32),
            pltpu.VMEM((B,), jnp.float32),
            pltpu.SemaphoreType.DMA,
        ],
        compiler_params=pltpu.CompilerParams(
            use_tc_tiling_on_sc=False,
            needs_layout_passes=False,
        ),
    )
    return kern(emb_u, idx_u)


def _mm_body(w_ref, p_ref, o_ref):
    o_ref[...] = lax.dot_general(
        w_ref[...],
        p_ref[...],
        dimension_numbers=(((0,), (0,)), ((), ())),
        preferred_element_type=jnp.float32,
    )


def _tc_matmul_t(ffw_t, pooled_t):
    return pl.pallas_call(
        _mm_body,
        grid=(NJ,),
        in_specs=[
            pl.BlockSpec((EMB, TN), lambda j: (0, j)),
            pl.BlockSpec((EMB, B), lambda j: (0, 0)),
        ],
        out_specs=pl.BlockSpec((TN, B), lambda j: (j, 0)),
        out_shape=jax.ShapeDtypeStruct((VOC, B), jnp.float32),
        compiler_params=pltpu.CompilerParams(
            dimension_semantics=("arbitrary",),
        ),
    )(ffw_t, pooled_t)


def kernel(inpt, emb_table, ffw_weight):
    idx_u = inpt.astype(jnp.int32).T
    pooled_t = _sc_pool_t(emb_table.T, idx_u)
    out_t = _tc_matmul_t(ffw_weight.T, pooled_t)
    return out_t.T
